# baseline (speedup 1.0000x reference)
import jax
import jax.numpy as jnp
from jax.experimental import pallas as pl
from jax.experimental.pallas import tpu as pltpu

N = 100000
STATIC_DIM = 128
DYNAMIC_DIM = 128
OUT_DIM = STATIC_DIM + DYNAMIC_DIM

CHUNK_SIZES = (2000, 6000) + (16000,) * 5 + (8000, 2000, 2000)
CHUNK_OFFS = tuple(sum(CHUNK_SIZES[:i]) for i in range(len(CHUNK_SIZES)))
assert sum(CHUNK_SIZES) == N
MAX_CHUNK = max(CHUNK_SIZES)
N_CHUNKS = len(CHUNK_SIZES)
N_SLOTS = 3


def _body(a_hbm, b_hbm, o_hbm, obuf, in_sems, out_sems):
    def in_copies(idx):
        s = idx % N_SLOTS
        rows = pl.ds(CHUNK_OFFS[idx], CHUNK_SIZES[idx])
        sub = pl.ds(0, CHUNK_SIZES[idx])
        return (
            pltpu.make_async_copy(
                a_hbm.at[rows, :],
                obuf.at[s, sub, pl.ds(0, STATIC_DIM)], in_sems.at[s, 0]),
            pltpu.make_async_copy(
                b_hbm.at[rows, :],
                obuf.at[s, sub, pl.ds(STATIC_DIM, DYNAMIC_DIM)],
                in_sems.at[s, 1]),
        )

    def out_copy(idx):
        s = idx % N_SLOTS
        rows = pl.ds(CHUNK_OFFS[idx], CHUNK_SIZES[idx])
        return pltpu.make_async_copy(
            obuf.at[s, pl.ds(0, CHUNK_SIZES[idx]), :], o_hbm.at[rows, :],
            out_sems.at[s])

    for idx in range(N_CHUNKS):
        if idx >= N_SLOTS:
            out_copy(idx - N_SLOTS).wait()
        for cp in in_copies(idx):
            cp.start()
        if idx >= 1:
            for cp in in_copies(idx - 1):
                cp.wait()
            out_copy(idx - 1).start()
    for cp in in_copies(N_CHUNKS - 1):
        cp.wait()
    out_copy(N_CHUNKS - 1).start()
    for idx in range(N_CHUNKS - N_SLOTS, N_CHUNKS):
        out_copy(idx).wait()


def kernel(static_emb, dynamic_emb):
    return pl.pallas_call(
        _body,
        in_specs=[
            pl.BlockSpec(memory_space=pltpu.MemorySpace.HBM),
            pl.BlockSpec(memory_space=pltpu.MemorySpace.HBM),
        ],
        out_specs=pl.BlockSpec(memory_space=pltpu.MemorySpace.HBM),
        out_shape=jax.ShapeDtypeStruct((N, OUT_DIM), jnp.float32),
        scratch_shapes=[
            pltpu.VMEM((N_SLOTS, MAX_CHUNK, OUT_DIM), jnp.float32),
            pltpu.SemaphoreType.DMA((N_SLOTS, 2)),
            pltpu.SemaphoreType.DMA((N_SLOTS,)),
        ],
    )(static_emb, dynamic_emb)


# speedup vs baseline: 1.0089x; 1.0089x over previous
import jax
import jax.numpy as jnp
from jax.experimental import pallas as pl
from jax.experimental.pallas import tpu as pltpu

N = 100000
STATIC_DIM = 128
DYNAMIC_DIM = 128
OUT_DIM = STATIC_DIM + DYNAMIC_DIM

CHUNK_SIZES = (2000, 6000) + (15000,) * 6 + (2000,)
CHUNK_OFFS = tuple(sum(CHUNK_SIZES[:i]) for i in range(len(CHUNK_SIZES)))
assert sum(CHUNK_SIZES) == N
MAX_CHUNK = max(CHUNK_SIZES)
N_CHUNKS = len(CHUNK_SIZES)
N_SLOTS = 3


def _body(a_hbm, b_hbm, o_hbm, obuf, in_sems, out_sems):
    def in_copies(idx):
        s = idx % N_SLOTS
        rows = pl.ds(CHUNK_OFFS[idx], CHUNK_SIZES[idx])
        sub = pl.ds(0, CHUNK_SIZES[idx])
        return (
            pltpu.make_async_copy(
                a_hbm.at[rows, :],
                obuf.at[s, sub, pl.ds(0, STATIC_DIM)], in_sems.at[s, 0]),
            pltpu.make_async_copy(
                b_hbm.at[rows, :],
                obuf.at[s, sub, pl.ds(STATIC_DIM, DYNAMIC_DIM)],
                in_sems.at[s, 1]),
        )

    def out_copy(idx):
        s = idx % N_SLOTS
        rows = pl.ds(CHUNK_OFFS[idx], CHUNK_SIZES[idx])
        return pltpu.make_async_copy(
            obuf.at[s, pl.ds(0, CHUNK_SIZES[idx]), :], o_hbm.at[rows, :],
            out_sems.at[s])

    for idx in range(N_CHUNKS):
        if idx >= N_SLOTS:
            out_copy(idx - N_SLOTS).wait()
        for cp in in_copies(idx):
            cp.start()
        if idx >= 1:
            for cp in in_copies(idx - 1):
                cp.wait()
            out_copy(idx - 1).start()
    for cp in in_copies(N_CHUNKS - 1):
        cp.wait()
    out_copy(N_CHUNKS - 1).start()
    for idx in range(N_CHUNKS - N_SLOTS, N_CHUNKS):
        out_copy(idx).wait()


def kernel(static_emb, dynamic_emb):
    return pl.pallas_call(
        _body,
        in_specs=[
            pl.BlockSpec(memory_space=pltpu.MemorySpace.HBM),
            pl.BlockSpec(memory_space=pltpu.MemorySpace.HBM),
        ],
        out_specs=pl.BlockSpec(memory_space=pltpu.MemorySpace.HBM),
        out_shape=jax.ShapeDtypeStruct((N, OUT_DIM), jnp.float32),
        scratch_shapes=[
            pltpu.VMEM((N_SLOTS, MAX_CHUNK, OUT_DIM), jnp.float32),
            pltpu.SemaphoreType.DMA((N_SLOTS, 2)),
            pltpu.SemaphoreType.DMA((N_SLOTS,)),
        ],
    )(static_emb, dynamic_emb)
